# baseline (speedup 1.0000x reference)
import functools

import jax
import jax.numpy as jnp
from jax.experimental import pallas as pl
from jax.experimental.pallas import tpu as pltpu

_INV_SQRT2 = 0.7071067811865476
_LN_EPS = 1e-5


def _yc_kernel(a_ref, b_ref, c_ref, out_ref, *, T):
    S = a_ref.shape[0]
    hi = jax.lax.Precision.HIGHEST
    statesT = jnp.broadcast_to(b_ref[...], (S, T))
    P = a_ref[...]
    shift = 1
    while shift < T:
        shifted = jnp.concatenate(
            [jnp.zeros((S, shift), jnp.float32), statesT[:, : T - shift]], axis=1
        )
        statesT = statesT + jax.lax.dot(
            P, shifted, precision=hi, preferred_element_type=jnp.float32
        )
        shift *= 2
        if shift < T:
            P = jax.lax.dot(P, P, precision=hi, preferred_element_type=jnp.float32)
    out_ref[...] = jax.lax.dot_general(
        c_ref[...], statesT, (((0,), (0,)), ((), ())),
        precision=hi, preferred_element_type=jnp.float32,
    )


def _fused_kernel(x_ref, yc_ref, d_ref, w_ref, bias_ref, out_ref, *, B, F):
    T = x_ref.shape[-1]
    xv = x_ref[...].reshape(B, F, T)
    t = xv * d_ref[...][None] + yc_ref[...][None]
    g = 0.5 * t * (1.0 + jax.lax.erf(t * _INV_SQRT2))
    o = xv + g
    mu = jnp.mean(o, axis=1, keepdims=True)
    m2 = jnp.mean(o * o, axis=1, keepdims=True)
    var = m2 - mu * mu
    rs = jax.lax.rsqrt(var + _LN_EPS)
    res = (o - mu) * rs * w_ref[...][None] + bias_ref[...][None]
    out_ref[...] = res.reshape(B * F, T)


def kernel(x, A, Bvec, Cmat, D, ln_w, ln_b):
    B, F, H, T = x.shape
    S = A.shape[0]

    yc = pl.pallas_call(
        functools.partial(_yc_kernel, T=T),
        out_shape=jax.ShapeDtypeStruct((F, T), jnp.float32),
        name="ssm_states_yc",
    )(A, Bvec.reshape(S, 1), Cmat)

    x2d = x.reshape(B * F, H * T)
    const_spec = pl.BlockSpec((F, 1), lambda h: (0, 0))
    out2d = pl.pallas_call(
        functools.partial(_fused_kernel, B=B, F=F),
        grid=(H,),
        in_specs=[
            pl.BlockSpec((B * F, T), lambda h: (0, h)),
            pl.BlockSpec((F, T), lambda h: (0, 0)),
            const_spec,
            const_spec,
            const_spec,
        ],
        out_specs=pl.BlockSpec((B * F, T), lambda h: (0, h)),
        out_shape=jax.ShapeDtypeStruct((B * F, H * T), x.dtype),
        compiler_params=pltpu.CompilerParams(
            dimension_semantics=("parallel",),
            vmem_limit_bytes=48 * 1024 * 1024,
        ),
        name="ssm_gelu_ln",
    )(x2d, yc, D.reshape(F, 1), ln_w.reshape(F, 1), ln_b.reshape(F, 1))
    return out2d.reshape(B, F, H, T)


# speedup vs baseline: 3.4856x; 3.4856x over previous
import functools

import jax
import jax.numpy as jnp
from jax.experimental import pallas as pl
from jax.experimental.pallas import tpu as pltpu

_INV_SQRT2 = 0.7071067811865476
_LN_EPS = 1e-5


def _yc_kernel(a_ref, b_ref, c_ref, out_ref, *, T):
    S = a_ref.shape[0]
    hi = jax.lax.Precision.HIGHEST
    statesT = jnp.broadcast_to(b_ref[...], (S, T))
    P = a_ref[...]
    shift = 1
    while shift < T:
        shifted = jnp.concatenate(
            [jnp.zeros((S, shift), jnp.float32), statesT[:, : T - shift]], axis=1
        )
        statesT = statesT + jax.lax.dot(
            P, shifted, precision=hi, preferred_element_type=jnp.float32
        )
        shift *= 2
        if shift < T:
            P = jax.lax.dot(P, P, precision=hi, preferred_element_type=jnp.float32)
    out_ref[...] = jax.lax.dot_general(
        c_ref[...], statesT, (((0,), (0,)), ((), ())),
        precision=hi, preferred_element_type=jnp.float32,
    )


def _fused_kernel(x_ref, yc_ref, d_ref, w_ref, bias_ref, out_ref):
    xv = x_ref[...]
    t = xv * d_ref[...] + yc_ref[...]
    g = 0.5 * t * (1.0 + jax.lax.erf(t * _INV_SQRT2))
    o = xv + g
    mu = jnp.mean(o, axis=1, keepdims=True)
    m2 = jnp.mean(o * o, axis=1, keepdims=True)
    var = m2 - mu * mu
    rs = jax.lax.rsqrt(var + _LN_EPS)
    out_ref[...] = (o - mu) * rs * w_ref[...] + bias_ref[...]


def kernel(x, A, Bvec, Cmat, D, ln_w, ln_b):
    B, F, H, T = x.shape
    S = A.shape[0]
    HB = 16

    yc = pl.pallas_call(
        functools.partial(_yc_kernel, T=T),
        out_shape=jax.ShapeDtypeStruct((F, T), jnp.float32),
        name="ssm_states_yc",
    )(A, Bvec.reshape(S, 1), Cmat)

    yc4 = yc.reshape(1, F, 1, T)
    d4 = D.reshape(1, F, 1, 1)
    w4 = ln_w.reshape(1, F, 1, 1)
    b4 = ln_b.reshape(1, F, 1, 1)

    HT = H // HB

    const_spec = pl.BlockSpec((1, F, 1, 1), lambda i: (0, 0, 0, 0))
    out = pl.pallas_call(
        _fused_kernel,
        grid=(B * HT,),
        in_specs=[
            pl.BlockSpec((1, F, HB, T), lambda i: (i // HT, 0, i % HT, 0)),
            pl.BlockSpec((1, F, 1, T), lambda i: (0, 0, 0, 0)),
            const_spec,
            const_spec,
            const_spec,
        ],
        out_specs=pl.BlockSpec((1, F, HB, T), lambda i: (i // HT, 0, i % HT, 0)),
        out_shape=jax.ShapeDtypeStruct(x.shape, x.dtype),
        compiler_params=pltpu.CompilerParams(
            dimension_semantics=("arbitrary",),
            vmem_limit_bytes=52 * 1024 * 1024,
        ),
        name="ssm_gelu_ln",
    )(x, yc4, d4, w4, b4)
    return out
